# baseline (speedup 1.0000x reference)
import jax
import jax.numpy as jnp
from jax import lax
from jax.experimental import pallas as pl
from jax.experimental.pallas import tpu as pltpu
from jax.experimental.pallas import tpu_sc as plsc

_B = 16384
_F = 64
_GLOBAL_MEAN = 3.5

_NC = 2
_NS = 16
_NW = _NC * _NS
_BPW = _B // _NW
_CH = 128
_NCHUNK = _BPW // _CH
_URING = 8
_IRING = 2


def _dot_body(su_hbm, siid_hbm, irow_hbm, uft_hbm, if2_hbm, dots_hbm,
              su_v, siid_v, irow_v, su_f, siid_f,
              uring_v, iring_v, out_v, usem, isem):
    wid = lax.axis_index("s") * _NC + lax.axis_index("c")
    base = wid * _BPW

    for j in range(_NCHUNK):
        sl = pl.ds(base + j * _CH, _CH)
        pltpu.sync_copy(su_hbm.at[sl], su_v.at[j])
        pltpu.sync_copy(siid_hbm.at[sl], siid_v.at[j])
        pltpu.sync_copy(irow_hbm.at[sl], irow_v.at[j])
    pltpu.sync_copy(su_hbm.at[pl.ds(base, _BPW)], su_f)
    pltpu.sync_copy(siid_hbm.at[pl.ds(base, _BPW)], siid_f)

    blk0 = lax.shift_right_logical(su_f[pl.ds(0, 16)][0], 7)
    blk_last = lax.shift_right_logical(su_f[pl.ds(_BPW - 16, 16)][15], 7)

    def fire_ublock(b):
        @pl.when(b <= blk_last)
        def _():
            off = pl.multiple_of(b * _CH, _CH)
            pltpu.async_copy(uft_hbm.at[:, pl.ds(off, _CH)],
                             uring_v.at[lax.rem(b, _URING)], usem)

    def fire_ichunk(c):
        @pl.when(c < _NCHUNK)
        def _():
            pltpu.async_copy(if2_hbm.at[irow_v.at[c]],
                             iring_v.at[lax.rem(c, _IRING)], isem)

    for k in range(_URING - 1):
        fire_ublock(blk0 + k)
    for c in range(_IRING - 1):
        fire_ichunk(c)

    def wait_ublock():
        pltpu.make_async_copy(uft_hbm.at[:, pl.ds(0, _CH)],
                              uring_v.at[0], usem).wait()

    def wait_ichunk():
        pltpu.make_async_copy(if2_hbm.at[irow_v.at[0]],
                              iring_v.at[0], isem).wait()

    iota = lax.iota(jnp.int32, 16)

    def group_body(g, carry):
        uhead, ihead = carry
        g0 = g * 16
        suvec = su_f[pl.ds(g0, 16)]
        sivec = siid_f[pl.ds(g0, 16)]
        j = lax.shift_right_logical(g, 3)
        islot = jnp.full((16,), lax.rem(j, _IRING), jnp.int32)

        def i_adv(c):
            wait_ichunk()
            fire_ichunk(c + _IRING - 1)
            return c + 1

        ihead = lax.while_loop(lambda c: c <= j, i_adv, ihead)

        accvec = jnp.zeros((16,), jnp.float32)
        for k in range(16):
            su_s = suvec[k]
            blk = lax.shift_right_logical(su_s, 7)

            def u_adv(c):
                wait_ublock()
                fire_ublock(c + _URING - 1)
                return c + 1

            uhead = lax.while_loop(lambda c: c <= blk, u_adv, uhead)

            uslot = jnp.full((16,), lax.rem(blk, _URING), jnp.int32)
            srow = jnp.full((16,), g0 + k - j * _CH, jnp.int32)
            col = jnp.full((16,), su_s & (_CH - 1), jnp.int32)
            ih64 = (sivec[k] & 1) * _F

            partial = jnp.zeros((16,), jnp.float32)
            for f0 in range(0, _F, 16):
                fvec = iota + f0
                u = plsc.load_gather(uring_v, [uslot, fvec, col])
                q = plsc.load_gather(iring_v, [islot, srow, fvec + ih64])
                partial = partial + u * q
            dotv = jnp.full((16,), jnp.sum(partial), jnp.float32)
            accvec = jnp.where(iota == k, dotv, accvec)

        out_v[pl.ds(g0, 16)] = accvec
        return (uhead, ihead)

    lax.fori_loop(0, _BPW // 16, group_body, (blk0, jnp.int32(0)))

    pltpu.sync_copy(out_v, dots_hbm.at[pl.ds(base, _BPW)])


def _bias_body(dots_hbm, su_hbm, siid_hbm, order_hbm, ub_hbm, ib_hbm,
               out_hbm, su_v, siid_v, order_v, ubias_v, ibias_v, out_v,
               bsem):
    wid = lax.axis_index("s") * _NC + lax.axis_index("c")
    base = wid * _BPW

    for j in range(_NCHUNK):
        sl = pl.ds(base + j * _CH, _CH)
        pltpu.sync_copy(su_hbm.at[sl], su_v.at[j])
        pltpu.sync_copy(siid_hbm.at[sl], siid_v.at[j])
        pltpu.sync_copy(order_hbm.at[sl], order_v.at[j])

    bias_copies = []
    for j in range(_NCHUNK):
        sl = pl.ds(j * _CH, _CH)
        bias_copies.append(
            pltpu.async_copy(ub_hbm.at[su_v.at[j]], ubias_v.at[sl], bsem))
        bias_copies.append(
            pltpu.async_copy(ib_hbm.at[siid_v.at[j]], ibias_v.at[sl], bsem))

    pltpu.sync_copy(dots_hbm.at[pl.ds(base, _BPW)], out_v)
    for c in bias_copies:
        c.wait()

    def bias_body(g, _):
        sl = pl.ds(g * 16, 16)
        out_v[sl] = out_v[sl] + ubias_v[sl] + ibias_v[sl] + _GLOBAL_MEAN
        return 0

    lax.fori_loop(0, _BPW // 16, bias_body, 0)

    out_copies = [
        pltpu.async_copy(out_v.at[pl.ds(j * _CH, _CH)],
                         out_hbm.at[order_v.at[j]], bsem)
        for j in range(_NCHUNK)
    ]
    for c in out_copies:
        c.wait()


@jax.jit
def _svd_predict(user_ids, item_ids, user_factors, item_factors,
                 user_bias, item_bias):
    uid = user_ids.astype(jnp.int32)
    iid = item_ids.astype(jnp.int32)
    order = jnp.argsort(uid)
    su = jnp.take(uid, order)
    siid = jnp.take(iid, order)

    mesh = plsc.VectorSubcoreMesh(core_axis_name="c", subcore_axis_name="s")
    run_dots = pl.kernel(
        _dot_body,
        out_type=jax.ShapeDtypeStruct((_B,), jnp.float32),
        mesh=mesh,
        scratch_types=[
            pltpu.VMEM((_NCHUNK, _CH), jnp.int32),
            pltpu.VMEM((_NCHUNK, _CH), jnp.int32),
            pltpu.VMEM((_NCHUNK, _CH), jnp.int32),
            pltpu.VMEM((_BPW,), jnp.int32),
            pltpu.VMEM((_BPW,), jnp.int32),
            pltpu.VMEM((_URING, _F, _CH), jnp.float32),
            pltpu.VMEM((_IRING, _CH, 2 * _F), jnp.float32),
            pltpu.VMEM((_BPW,), jnp.float32),
            pltpu.SemaphoreType.DMA,
            pltpu.SemaphoreType.DMA,
        ],
        compiler_params=pltpu.CompilerParams(needs_layout_passes=False),
    )
    run_bias = pl.kernel(
        _bias_body,
        out_type=jax.ShapeDtypeStruct((_B,), jnp.float32),
        mesh=mesh,
        scratch_types=[
            pltpu.VMEM((_NCHUNK, _CH), jnp.int32),
            pltpu.VMEM((_NCHUNK, _CH), jnp.int32),
            pltpu.VMEM((_NCHUNK, _CH), jnp.int32),
            pltpu.VMEM((_BPW,), jnp.float32),
            pltpu.VMEM((_BPW,), jnp.float32),
            pltpu.VMEM((_BPW,), jnp.float32),
            pltpu.SemaphoreType.DMA,
        ],
        compiler_params=pltpu.CompilerParams(needs_layout_passes=False),
    )
    dots = run_dots(su, siid, siid >> 1,
                    user_factors.T, item_factors.reshape(-1, 2 * _F))
    return run_bias(dots, su, siid, order.astype(jnp.int32),
                    user_bias.T.reshape(-1), item_bias.T.reshape(-1))


def kernel(user_ids, item_ids, user_factors, item_factors, user_bias,
           item_bias):
    return _svd_predict(user_ids, item_ids, user_factors, item_factors,
                        user_bias, item_bias)


# speedup vs baseline: 1.0189x; 1.0189x over previous
import jax
import jax.numpy as jnp
from jax import lax
from jax.experimental import pallas as pl
from jax.experimental.pallas import tpu as pltpu
from jax.experimental.pallas import tpu_sc as plsc

_B = 16384
_F = 64
_GLOBAL_MEAN = 3.5

_NC = 2
_NS = 16
_NW = _NC * _NS
_BPW = _B // _NW
_CH = 128
_NCHUNK = _BPW // _CH
_URING = 8
_IRING = 2


def _body(su_hbm, siid_hbm, irow_hbm, order_hbm,
          uft_hbm, if2_hbm, ub_hbm, ib_hbm, out_hbm,
          su_v, siid_v, irow_v, order_v, su_f, siid_f,
          uring_v, iring_v, ubias_v, ibias_v, out_v,
          usem, isem, bsem):
    wid = lax.axis_index("s") * _NC + lax.axis_index("c")
    base = wid * _BPW

    for j in range(_NCHUNK):
        sl = pl.ds(base + j * _CH, _CH)
        pltpu.sync_copy(su_hbm.at[sl], su_v.at[j])
        pltpu.sync_copy(siid_hbm.at[sl], siid_v.at[j])
        pltpu.sync_copy(irow_hbm.at[sl], irow_v.at[j])
        pltpu.sync_copy(order_hbm.at[sl], order_v.at[j])
    pltpu.sync_copy(su_hbm.at[pl.ds(base, _BPW)], su_f)
    pltpu.sync_copy(siid_hbm.at[pl.ds(base, _BPW)], siid_f)

    bias_copies = []
    for j in range(_NCHUNK):
        sl = pl.ds(j * _CH, _CH)
        bias_copies.append(
            pltpu.async_copy(ub_hbm.at[su_v.at[j]], ubias_v.at[sl], bsem))
        bias_copies.append(
            pltpu.async_copy(ib_hbm.at[siid_v.at[j]], ibias_v.at[sl], bsem))

    blk0 = lax.shift_right_logical(su_f[pl.ds(0, 16)][0], 7)
    blk_last = lax.shift_right_logical(su_f[pl.ds(_BPW - 16, 16)][15], 7)

    def fire_ublock(b):
        @pl.when(b <= blk_last)
        def _():
            off = pl.multiple_of(b * _CH, _CH)
            pltpu.async_copy(uft_hbm.at[:, pl.ds(off, _CH)],
                             uring_v.at[lax.rem(b, _URING)], usem)

    def fire_ichunk(c):
        @pl.when(c < _NCHUNK)
        def _():
            pltpu.async_copy(if2_hbm.at[irow_v.at[c]],
                             iring_v.at[lax.rem(c, _IRING)], isem)

    for k in range(_URING - 1):
        fire_ublock(blk0 + k)
    for c in range(_IRING - 1):
        fire_ichunk(c)

    def wait_ublock():
        pltpu.make_async_copy(uft_hbm.at[:, pl.ds(0, _CH)],
                              uring_v.at[0], usem).wait()

    def wait_ichunk():
        pltpu.make_async_copy(if2_hbm.at[irow_v.at[0]],
                              iring_v.at[0], isem).wait()

    iota = lax.iota(jnp.int32, 16)

    def group_body(g, carry):
        uhead, ihead = carry
        g0 = g * 16
        suvec = su_f[pl.ds(g0, 16)]
        sivec = siid_f[pl.ds(g0, 16)]
        j = lax.shift_right_logical(g, 3)
        islot = jnp.full((16,), lax.rem(j, _IRING), jnp.int32)

        def i_adv(c):
            wait_ichunk()
            fire_ichunk(c + _IRING - 1)
            return c + 1

        ihead = lax.while_loop(lambda c: c <= j, i_adv, ihead)

        accvec = jnp.zeros((16,), jnp.float32)
        for k in range(16):
            su_s = suvec[k]
            blk = lax.shift_right_logical(su_s, 7)

            def u_adv(c):
                wait_ublock()
                fire_ublock(c + _URING - 1)
                return c + 1

            uhead = lax.while_loop(lambda c: c <= blk, u_adv, uhead)

            uslot = jnp.full((16,), lax.rem(blk, _URING), jnp.int32)
            srow = jnp.full((16,), g0 + k - j * _CH, jnp.int32)
            col = jnp.full((16,), su_s & (_CH - 1), jnp.int32)
            ih64 = (sivec[k] & 1) * _F

            partial = jnp.zeros((16,), jnp.float32)
            for f0 in range(0, _F, 16):
                fvec = iota + f0
                u = plsc.load_gather(uring_v, [uslot, fvec, col])
                q = plsc.load_gather(iring_v, [islot, srow, fvec + ih64])
                partial = partial + u * q
            dotv = jnp.full((16,), jnp.sum(partial), jnp.float32)
            accvec = jnp.where(iota == k, dotv, accvec)

        out_v[pl.ds(g0, 16)] = accvec
        return (uhead, ihead)

    lax.fori_loop(0, _BPW // 16, group_body, (blk0, jnp.int32(0)))

    for c in bias_copies:
        c.wait()

    def bias_body(g, _):
        sl = pl.ds(g * 16, 16)
        out_v[sl] = out_v[sl] + ubias_v[sl] + ibias_v[sl] + _GLOBAL_MEAN
        return 0

    lax.fori_loop(0, _BPW // 16, bias_body, 0)

    out_copies = [
        pltpu.async_copy(out_v.at[pl.ds(j * _CH, _CH)],
                         out_hbm.at[order_v.at[j]], bsem)
        for j in range(_NCHUNK)
    ]
    for c in out_copies:
        c.wait()


@jax.jit
def _svd_predict(user_ids, item_ids, user_factors, item_factors,
                 user_bias, item_bias):
    uid = user_ids.astype(jnp.int32)
    iid = item_ids.astype(jnp.int32)
    order = jnp.argsort(uid)
    su = jnp.take(uid, order)
    siid = jnp.take(iid, order)

    mesh = plsc.VectorSubcoreMesh(core_axis_name="c", subcore_axis_name="s")
    run = pl.kernel(
        _body,
        out_type=jax.ShapeDtypeStruct((_B,), jnp.float32),
        mesh=mesh,
        scratch_types=[
            pltpu.VMEM((_NCHUNK, _CH), jnp.int32),
            pltpu.VMEM((_NCHUNK, _CH), jnp.int32),
            pltpu.VMEM((_NCHUNK, _CH), jnp.int32),
            pltpu.VMEM((_NCHUNK, _CH), jnp.int32),
            pltpu.VMEM((_BPW,), jnp.int32),
            pltpu.VMEM((_BPW,), jnp.int32),
            pltpu.VMEM((_URING, _F, _CH), jnp.float32),
            pltpu.VMEM((_IRING, _CH, 2 * _F), jnp.float32),
            pltpu.VMEM((_BPW,), jnp.float32),
            pltpu.VMEM((_BPW,), jnp.float32),
            pltpu.VMEM((_BPW,), jnp.float32),
            pltpu.SemaphoreType.DMA,
            pltpu.SemaphoreType.DMA,
            pltpu.SemaphoreType.DMA,
        ],
        compiler_params=pltpu.CompilerParams(needs_layout_passes=False),
    )
    return run(su, siid, siid >> 1, order.astype(jnp.int32),
               user_factors.T, item_factors.reshape(-1, 2 * _F),
               user_bias.T.reshape(-1), item_bias.T.reshape(-1))


def kernel(user_ids, item_ids, user_factors, item_factors, user_bias,
           item_bias):
    return _svd_predict(user_ids, item_ids, user_factors, item_factors,
                        user_bias, item_bias)
